# Initial kernel scaffold; baseline (speedup 1.0000x reference)
import jax
import jax.numpy as jnp
from jax.experimental import pallas as pl


def kernel(x, edge_index, W1, b1, W2, b2):
    raise NotImplementedError("write your pallas kernel here")


# speedup vs baseline: 18.6602x; 18.6602x over previous
import functools

import jax
import jax.numpy as jnp
from jax import lax
from jax.experimental import pallas as pl
from jax.experimental.pallas import tpu as pltpu
from jax.experimental.pallas import tpu_sc as plsc

NC = 2
NS = 16
NW = NC * NS
LANES = 16
K = 128

F32 = jnp.float32
I32 = jnp.int32


def _mesh():
    return plsc.VectorSubcoreMesh(core_axis_name="c", subcore_axis_name="s")


def _sc_degrees(dst2, P, EPW):
    @functools.partial(
        pl.kernel,
        out_type=jax.ShapeDtypeStruct((NW, P), F32),
        mesh=_mesh(),
        compiler_params=pltpu.CompilerParams(needs_layout_passes=False),
        scratch_types=[
            pltpu.VMEM((EPW,), I32),
            pltpu.VMEM((P,), F32),
        ],
    )
    def k(dst_hbm, out_hbm, didx_v, acc_v):
        c = lax.axis_index("c")
        s = lax.axis_index("s")
        w = c * NS + s

        def zero(i, _):
            acc_v[pl.ds(i * LANES, LANES)] = jnp.zeros((LANES,), F32)
            return 0

        lax.fori_loop(0, P // LANES, zero, 0)
        pltpu.sync_copy(dst_hbm.at[w], didx_v)
        ones16 = jnp.ones((LANES,), F32)

        def body(j, _):
            idx = didx_v[pl.ds(j * LANES, LANES)]
            plsc.addupdate_scatter(acc_v, [idx], ones16)
            return 0

        lax.fori_loop(0, EPW // LANES, body, 0)
        pltpu.sync_copy(acc_v, out_hbm.at[w])

    return k


def _sc_rows(P, NCHUNK):
    STRIPE = P // NS

    @functools.partial(
        pl.kernel,
        out_type=jax.ShapeDtypeStruct((NC, P, 128), F32),
        mesh=_mesh(),
        compiler_params=pltpu.CompilerParams(needs_layout_passes=False),
        scratch_types=[
            pltpu.VMEM((NCHUNK, K), I32),
            pltpu.VMEM((NCHUNK, K), I32),
            pltpu.VMEM((K, 128), F32),
            pltpu.VMEM_SHARED((P, 128), F32),
            pltpu.SemaphoreType.DMA,
        ],
    )
    def k(g_hbm, src_hbm, dst_hbm, out_hbm, sidx_v, didx_v, rows_v, acc_sh, sem):
        c = lax.axis_index("c")
        s = lax.axis_index("s")
        w = c * NS + s
        pltpu.sync_copy(src_hbm.at[w], sidx_v)
        pltpu.sync_copy(dst_hbm.at[w], didx_v)

        zero16 = jnp.zeros((LANES,), F32)

        def zrow(r, _):
            for j in range(128 // LANES):
                rows_v[r, pl.ds(j * LANES, LANES)] = zero16
            return 0

        lax.fori_loop(0, K, zrow, 0)
        for t in range(STRIPE // K):
            pltpu.sync_copy(rows_v, acc_sh.at[pl.ds(s * STRIPE + t * K, K)])
        plsc.subcore_barrier()

        def chunk(i, _):
            pltpu.async_copy(g_hbm.at[sidx_v.at[i]], rows_v, sem).wait()
            pltpu.sync_copy(rows_v, acc_sh.at[didx_v.at[i]], add=True)
            return 0

        lax.fori_loop(0, NCHUNK, chunk, 0)
        plsc.subcore_barrier()
        for t in range(STRIPE // K):
            sl = pl.ds(s * STRIPE + t * K, K)
            pltpu.sync_copy(acc_sh.at[sl], rows_v)
            pltpu.sync_copy(rows_v, out_hbm.at[c, sl])

    return k


def _sc_scalars(P, EPW):
    @functools.partial(
        pl.kernel,
        out_type=jax.ShapeDtypeStruct((NW, P), F32),
        mesh=_mesh(),
        compiler_params=pltpu.CompilerParams(needs_layout_passes=False),
        scratch_types=[
            pltpu.VMEM((EPW,), I32),
            pltpu.VMEM((EPW,), I32),
            pltpu.VMEM((P,), F32),
            pltpu.VMEM((P,), F32),
        ],
    )
    def k(zs_hbm, src_hbm, dst_hbm, out_hbm, sidx_v, didx_v, zs_v, acc_v):
        c = lax.axis_index("c")
        s = lax.axis_index("s")
        w = c * NS + s
        pltpu.sync_copy(zs_hbm, zs_v)
        pltpu.sync_copy(src_hbm.at[w], sidx_v)
        pltpu.sync_copy(dst_hbm.at[w], didx_v)

        def zero(i, _):
            acc_v[pl.ds(i * LANES, LANES)] = jnp.zeros((LANES,), F32)
            return 0

        lax.fori_loop(0, P // LANES, zero, 0)

        def body(j, _):
            sl = pl.ds(j * LANES, LANES)
            vals = plsc.load_gather(zs_v, [sidx_v[sl]])
            plsc.addupdate_scatter(acc_v, [didx_v[sl]], vals)
            return 0

        lax.fori_loop(0, EPW // LANES, body, 0)
        pltpu.sync_copy(acc_v, out_hbm.at[w])

    return k


def _tc_g(deg3, x_pad, W1, P, BR):
    grid = (P // BR,)

    def body(deg_ref, x_ref, w1_ref, g_ref):
        deg = jnp.sum(deg_ref[...], axis=0) + 1.0
        dis = lax.rsqrt(deg)
        h = jnp.dot(x_ref[...], w1_ref[...], preferred_element_type=F32)
        g_ref[...] = dis * h

    return pl.pallas_call(
        body,
        grid=grid,
        in_specs=[
            pl.BlockSpec((NW, BR, 1), lambda i: (0, i, 0)),
            pl.BlockSpec((BR, 128), lambda i: (i, 0)),
            pl.BlockSpec((128, 128), lambda i: (0, 0)),
        ],
        out_specs=pl.BlockSpec((BR, 128), lambda i: (i, 0)),
        out_shape=jax.ShapeDtypeStruct((P, 128), F32),
    )(deg3, x_pad, W1)


def _tc_zs(acc_part, g, deg3, b1r, w2r, P, BR):
    grid = (P // BR,)

    def body(acc_ref, g_ref, deg_ref, b1_ref, w2_ref, zs_ref):
        acc = acc_ref[0] + acc_ref[1]
        deg = jnp.sum(deg_ref[...], axis=0) + 1.0
        dis = lax.rsqrt(deg)
        h1 = jnp.maximum(dis * (acc + g_ref[...]) + b1_ref[...], 0.0)
        z = jnp.sum(h1 * w2_ref[...], axis=1, keepdims=True)
        zs_ref[...] = dis * z

    return pl.pallas_call(
        body,
        grid=grid,
        in_specs=[
            pl.BlockSpec((NC, BR, 128), lambda i: (0, i, 0)),
            pl.BlockSpec((BR, 128), lambda i: (i, 0)),
            pl.BlockSpec((NW, BR, 1), lambda i: (0, i, 0)),
            pl.BlockSpec((1, 128), lambda i: (0, 0)),
            pl.BlockSpec((1, 128), lambda i: (0, 0)),
        ],
        out_specs=pl.BlockSpec((BR, 1), lambda i: (i, 0)),
        out_shape=jax.ShapeDtypeStruct((P, 1), F32),
    )(acc_part, g, deg3, b1r, w2r)


def _tc_out(sacc2, zs2, deg2, b2r, P):
    R = P // 128

    def body(sacc_ref, zs_ref, deg_ref, b2_ref, out_ref):
        sacc = jnp.sum(sacc_ref[...], axis=0)
        deg = jnp.sum(deg_ref[...], axis=0) + 1.0
        dis = lax.rsqrt(deg)
        out_ref[...] = dis * (sacc + zs_ref[...]) + b2_ref[0, 0]

    return pl.pallas_call(
        body,
        out_shape=jax.ShapeDtypeStruct((R, 128), F32),
    )(sacc2, zs2, deg2, b2r)


def kernel(x, edge_index, W1, b1, W2, b2):
    N, D = x.shape
    H = W1.shape[1]
    E = edge_index.shape[1]
    src = edge_index[0].astype(I32)
    dst = edge_index[1].astype(I32)

    P = -(-(N + 1) // (NS * K)) * (NS * K)
    EPW = -(-E // (NW * K)) * K
    EPAD = EPW * NW
    NCHUNK = EPW // K
    BR = 512

    x_pad = jnp.zeros((P, D), F32).at[:N].set(x)
    src_p = jnp.full((EPAD,), N, I32).at[:E].set(src)
    dst_p = jnp.full((EPAD,), N, I32).at[:E].set(dst)
    src2 = src_p.reshape(NW, EPW)
    dst2 = dst_p.reshape(NW, EPW)
    src3 = src_p.reshape(NW, NCHUNK, K)
    dst3 = dst_p.reshape(NW, NCHUNK, K)

    deg_part = _sc_degrees(dst2, P, EPW)(dst2)
    deg3 = deg_part.reshape(NW, P, 1)

    g = _tc_g(deg3, x_pad, W1, P, BR)
    acc_part = _sc_rows(P, NCHUNK)(g, src3, dst3)

    b1r = b1.reshape(1, H)
    w2r = W2.reshape(1, H)
    zs = _tc_zs(acc_part, g, deg3, b1r, w2r, P, BR)

    sacc_part = _sc_scalars(P, EPW)(zs.reshape(P), src2, dst2)

    out2 = _tc_out(
        sacc_part.reshape(NW, P // 128, 128),
        zs.reshape(P // 128, 128),
        deg_part.reshape(NW, P // 128, 128),
        b2.reshape(1, 1),
        P,
    )
    return out2.reshape(-1)[:N]
